# Initial kernel scaffold; baseline (speedup 1.0000x reference)
import jax
import jax.numpy as jnp
from jax.experimental import pallas as pl


def kernel(visual_embed, textual_embed, labels, W):
    raise NotImplementedError("write your pallas kernel here")


# speedup vs baseline: 1.8002x; 1.8002x over previous
import jax
import jax.numpy as jnp
from jax.experimental import pallas as pl
from jax.experimental.pallas import tpu as pltpu

SCALE = 28.0
ALPHA = 0.6
BETA = 0.4
SCALE_POS = 10.0
SCALE_NEG = 40.0
NUM_CLASSES = 11003
FEATURE_SIZE = 2048
BATCH = 1024

_CB = 512
_NBJ = 11
_NPAD = 2 * _NBJ * _CB


def _norm_body(e_ref, o_ref):
    x = e_ref[...]
    ssq = jnp.sum(x * x, axis=1, keepdims=True)
    o_ref[...] = (x * jax.lax.rsqrt(ssq)).astype(jnp.bfloat16)


def _main_body(en_ref, w_ref, labb_ref, se_ref, la_ref):
    i = pl.program_id(0)
    j = pl.program_id(1)

    @pl.when(j == 0)
    def _():
        se_ref[...] = jnp.zeros(se_ref.shape, se_ref.dtype)
        la_ref[...] = jnp.zeros(la_ref.shape, la_ref.dtype)

    w = w_ref[...]
    ssq = jnp.sum(w * w, axis=0, keepdims=True)
    col0 = (i * _NBJ + j) * _CB
    cid1 = col0 + jax.lax.broadcasted_iota(jnp.int32, (1, _CB), 1)
    ok = (cid1 < NUM_CLASSES) & (ssq > 0.0)
    scale = jnp.where(ok, SCALE * jax.lax.rsqrt(ssq), 0.0)

    raw = jnp.dot(en_ref[...], w.astype(jnp.bfloat16),
                  preferred_element_type=jnp.float32)
    logits = raw * scale
    shifted = jnp.where(ok, logits - SCALE, -1e30)
    ex = jnp.exp(shifted)

    cid = col0 + jax.lax.broadcasted_iota(jnp.int32, (2 * BATCH, _CB), 1)
    labm = pltpu.repeat(labb_ref[...], _CB // 128, axis=1) == cid
    lv = jnp.where(labm, logits, 0.0)

    se_acc = se_ref[0]
    la_acc = la_ref[0]
    for k in range(_CB // 128):
        sl = slice(k * 128, (k + 1) * 128)
        se_acc = se_acc + ex[:, sl]
        la_acc = la_acc + lv[:, sl]
    se_ref[0] = se_acc
    la_ref[0] = la_acc


def _sim_body(v_ref, t_ref, labv_ref, labr_ref, ga_ref):
    j = pl.program_id(1)

    @pl.when(j == 0)
    def _():
        ga_ref[...] = jnp.zeros(ga_ref.shape, ga_ref.dtype)

    sim = jax.lax.dot_general(v_ref[...], t_ref[...],
                              (((1,), (1,)), ((), ())),
                              preferred_element_type=jnp.float32)
    posm = pltpu.repeat(labv_ref[...], BATCH // 128, axis=1) == \
        jnp.broadcast_to(labr_ref[...], sim.shape)
    x = jnp.where(posm, -SCALE_POS * (sim - ALPHA), SCALE_NEG * (sim - BETA))
    pp = jnp.maximum(x, 0.0) + jnp.log1p(jnp.exp(-jnp.abs(x)))
    prow = jnp.sum(pp, axis=0, keepdims=True)
    acc = ga_ref[0]
    for k in range(BATCH // 128):
        acc = acc + prow[:, k * 128:(k + 1) * 128]
    ga_ref[0] = acc


def _final_body(se_ref, la_ref, ga_ref, o_ref):
    s = se_ref[0] + se_ref[1]
    srow = jnp.sum(s, axis=1, keepdims=True)
    suml = jnp.sum(jnp.log(srow))
    labt = jnp.sum(la_ref[0] + la_ref[1])
    inst = (suml - labt) / BATCH + 2.0 * SCALE
    ga = 2.0 * jnp.sum(ga_ref[...]) / BATCH
    o_ref[0] = inst
    o_ref[1] = ga


def kernel(visual_embed, textual_embed, labels, W):
    labels = labels.astype(jnp.int32)
    E = jnp.concatenate([visual_embed, textual_embed], axis=0)
    lab2 = jnp.concatenate([labels, labels], axis=0)
    labb = jnp.broadcast_to(lab2[:, None], (2 * BATCH, 128))
    labv = jnp.broadcast_to(labels[:, None], (BATCH, 128))
    labr = labels[None, :]

    En = pl.pallas_call(
        _norm_body,
        grid=(2,),
        in_specs=[pl.BlockSpec((BATCH, FEATURE_SIZE), lambda i: (i, 0))],
        out_specs=pl.BlockSpec((BATCH, FEATURE_SIZE), lambda i: (i, 0)),
        out_shape=jax.ShapeDtypeStruct((2 * BATCH, FEATURE_SIZE), jnp.bfloat16),
        compiler_params=pltpu.CompilerParams(
            dimension_semantics=("parallel",)),
        name="normalize_embed",
    )(E)

    se, la = pl.pallas_call(
        _main_body,
        grid=(2, _NBJ),
        in_specs=[
            pl.BlockSpec((2 * BATCH, FEATURE_SIZE), lambda i, j: (0, 0)),
            pl.BlockSpec((FEATURE_SIZE, _CB), lambda i, j: (0, i * _NBJ + j)),
            pl.BlockSpec((2 * BATCH, 128), lambda i, j: (0, 0)),
        ],
        out_specs=[
            pl.BlockSpec((1, 2 * BATCH, 128), lambda i, j: (i, 0, 0)),
            pl.BlockSpec((1, 2 * BATCH, 128), lambda i, j: (i, 0, 0)),
        ],
        out_shape=[
            jax.ShapeDtypeStruct((2, 2 * BATCH, 128), jnp.float32),
            jax.ShapeDtypeStruct((2, 2 * BATCH, 128), jnp.float32),
        ],
        compiler_params=pltpu.CompilerParams(
            dimension_semantics=("parallel", "arbitrary"),
            vmem_limit_bytes=100 * 1024 * 1024),
        name="cosine_ce_partials",
    )(En, W, labb)

    ga_part = pl.pallas_call(
        _sim_body,
        grid=(2, 4),
        in_specs=[
            pl.BlockSpec((128, FEATURE_SIZE), lambda i, j: (i * 4 + j, 0)),
            pl.BlockSpec((BATCH, FEATURE_SIZE), lambda i, j: (1, 0)),
            pl.BlockSpec((128, 128), lambda i, j: (i * 4 + j, 0)),
            pl.BlockSpec((1, BATCH), lambda i, j: (0, 0)),
        ],
        out_specs=pl.BlockSpec((1, 1, 128), lambda i, j: (i, 0, 0)),
        out_shape=jax.ShapeDtypeStruct((2, 1, 128), jnp.float32),
        compiler_params=pltpu.CompilerParams(
            dimension_semantics=("parallel", "arbitrary"),
            vmem_limit_bytes=100 * 1024 * 1024),
        name="global_align_partials",
    )(En, En, labv, labr)

    out = pl.pallas_call(
        _final_body,
        in_specs=[
            pl.BlockSpec(memory_space=pltpu.VMEM),
            pl.BlockSpec(memory_space=pltpu.VMEM),
            pl.BlockSpec(memory_space=pltpu.VMEM),
        ],
        out_specs=pl.BlockSpec(memory_space=pltpu.SMEM),
        out_shape=jax.ShapeDtypeStruct((2,), jnp.float32),
        name="loss_combine",
    )(se, la, ga_part)
    return out


# speedup vs baseline: 2.3128x; 1.2848x over previous
import jax
import jax.numpy as jnp
from jax.experimental import pallas as pl
from jax.experimental.pallas import tpu as pltpu

SCALE = 28.0
ALPHA = 0.6
BETA = 0.4
SCALE_POS = 10.0
SCALE_NEG = 40.0
NUM_CLASSES = 11003
FEATURE_SIZE = 2048
BATCH = 1024

_CB = 512
_NBJ = 11
_NPAD = 2 * _NBJ * _CB


_E8 = 16.0


def _norm_body(e_ref, o8_ref, o16_ref):
    x = e_ref[...]
    n = x * jax.lax.rsqrt(jnp.sum(x * x, axis=1, keepdims=True))
    o8_ref[...] = (n * _E8).astype(jnp.float8_e4m3fn)
    o16_ref[...] = n.astype(jnp.bfloat16)


def _main_body(en_ref, w_ref, labb_ref, se_ref, la_ref):
    i = pl.program_id(0)
    j = pl.program_id(1)

    @pl.when(j == 0)
    def _():
        se_ref[...] = jnp.zeros(se_ref.shape, se_ref.dtype)
        la_ref[...] = jnp.zeros(la_ref.shape, la_ref.dtype)

    w = w_ref[...]
    ssq = jnp.sum(w * w, axis=0, keepdims=True)
    col0 = (i * _NBJ + j) * _CB
    cid1 = col0 + jax.lax.broadcasted_iota(jnp.int32, (1, _CB), 1)
    ok = (cid1 < NUM_CLASSES) & (ssq > 0.0)
    scale = jnp.where(ok, (SCALE / _E8) * jax.lax.rsqrt(ssq), 0.0)

    raw = jnp.dot(en_ref[...], w.astype(jnp.float8_e4m3fn),
                  preferred_element_type=jnp.float32)
    logits = raw * scale
    shifted = jnp.where(ok, logits - SCALE, -1e30)
    ex = jnp.exp(shifted)

    cid = col0 + jax.lax.broadcasted_iota(jnp.int32, (2 * BATCH, _CB), 1)
    labm = pltpu.repeat(labb_ref[...], _CB // 128, axis=1) == cid
    lv = jnp.where(labm, logits, 0.0)

    se_acc = se_ref[0]
    la_acc = la_ref[0]
    for k in range(_CB // 128):
        sl = slice(k * 128, (k + 1) * 128)
        se_acc = se_acc + ex[:, sl]
        la_acc = la_acc + lv[:, sl]
    se_ref[0] = se_acc
    la_ref[0] = la_acc


def _sim_body(v_ref, t_ref, labv_ref, labr_ref, ga_ref):
    j = pl.program_id(1)

    @pl.when(j == 0)
    def _():
        ga_ref[...] = jnp.zeros(ga_ref.shape, ga_ref.dtype)

    sim = jax.lax.dot_general(v_ref[...], t_ref[...],
                              (((1,), (1,)), ((), ())),
                              preferred_element_type=jnp.float32)
    posm = pltpu.repeat(labv_ref[...], BATCH // 128, axis=1) == \
        jnp.broadcast_to(labr_ref[...], sim.shape)
    x = jnp.where(posm, -SCALE_POS * (sim - ALPHA), SCALE_NEG * (sim - BETA))
    pp = jnp.maximum(x, 0.0) + jnp.log1p(jnp.exp(-jnp.abs(x)))
    prow = jnp.sum(pp, axis=0, keepdims=True)
    acc = ga_ref[0]
    for k in range(BATCH // 128):
        acc = acc + prow[:, k * 128:(k + 1) * 128]
    ga_ref[0] = acc


def _final_body(se_ref, la_ref, ga_ref, o_ref):
    s = se_ref[0] + se_ref[1]
    srow = jnp.sum(s, axis=1, keepdims=True)
    suml = jnp.sum(jnp.log(srow))
    labt = jnp.sum(la_ref[0] + la_ref[1])
    inst = (suml - labt) / BATCH + 2.0 * SCALE
    ga = 2.0 * jnp.sum(ga_ref[...]) / BATCH
    o_ref[0] = inst
    o_ref[1] = ga


def kernel(visual_embed, textual_embed, labels, W):
    labels = labels.astype(jnp.int32)
    E = jnp.concatenate([visual_embed, textual_embed], axis=0)
    lab2 = jnp.concatenate([labels, labels], axis=0)
    labb = jnp.broadcast_to(lab2[:, None], (2 * BATCH, 128))
    labv = jnp.broadcast_to(labels[:, None], (BATCH, 128))
    labr = labels[None, :]

    En8, En = pl.pallas_call(
        _norm_body,
        grid=(2,),
        in_specs=[pl.BlockSpec((BATCH, FEATURE_SIZE), lambda i: (i, 0))],
        out_specs=[
            pl.BlockSpec((BATCH, FEATURE_SIZE), lambda i: (i, 0)),
            pl.BlockSpec((BATCH, FEATURE_SIZE), lambda i: (i, 0)),
        ],
        out_shape=[
            jax.ShapeDtypeStruct((2 * BATCH, FEATURE_SIZE), jnp.float8_e4m3fn),
            jax.ShapeDtypeStruct((2 * BATCH, FEATURE_SIZE), jnp.bfloat16),
        ],
        compiler_params=pltpu.CompilerParams(
            dimension_semantics=("arbitrary",)),
        name="normalize_embed",
    )(E)

    se, la = pl.pallas_call(
        _main_body,
        grid=(2, _NBJ),
        in_specs=[
            pl.BlockSpec((2 * BATCH, FEATURE_SIZE), lambda i, j: (0, 0)),
            pl.BlockSpec((FEATURE_SIZE, _CB), lambda i, j: (0, i * _NBJ + j)),
            pl.BlockSpec((2 * BATCH, 128), lambda i, j: (0, 0)),
        ],
        out_specs=[
            pl.BlockSpec((1, 2 * BATCH, 128), lambda i, j: (i, 0, 0)),
            pl.BlockSpec((1, 2 * BATCH, 128), lambda i, j: (i, 0, 0)),
        ],
        out_shape=[
            jax.ShapeDtypeStruct((2, 2 * BATCH, 128), jnp.float32),
            jax.ShapeDtypeStruct((2, 2 * BATCH, 128), jnp.float32),
        ],
        compiler_params=pltpu.CompilerParams(
            dimension_semantics=("arbitrary", "arbitrary"),
            vmem_limit_bytes=100 * 1024 * 1024),
        name="cosine_ce_partials",
    )(En8, W, labb)

    ga_part = pl.pallas_call(
        _sim_body,
        grid=(2, 4),
        in_specs=[
            pl.BlockSpec((128, FEATURE_SIZE), lambda i, j: (i * 4 + j, 0)),
            pl.BlockSpec((BATCH, FEATURE_SIZE), lambda i, j: (1, 0)),
            pl.BlockSpec((128, 128), lambda i, j: (i * 4 + j, 0)),
            pl.BlockSpec((1, BATCH), lambda i, j: (0, 0)),
        ],
        out_specs=pl.BlockSpec((1, 1, 128), lambda i, j: (i, 0, 0)),
        out_shape=jax.ShapeDtypeStruct((2, 1, 128), jnp.float32),
        compiler_params=pltpu.CompilerParams(
            dimension_semantics=("arbitrary", "arbitrary"),
            vmem_limit_bytes=100 * 1024 * 1024),
        name="global_align_partials",
    )(En, En, labv, labr)

    out = pl.pallas_call(
        _final_body,
        in_specs=[
            pl.BlockSpec(memory_space=pltpu.VMEM),
            pl.BlockSpec(memory_space=pltpu.VMEM),
            pl.BlockSpec(memory_space=pltpu.VMEM),
        ],
        out_specs=pl.BlockSpec(memory_space=pltpu.SMEM),
        out_shape=jax.ShapeDtypeStruct((2,), jnp.float32),
        name="loss_combine",
    )(se, la, ga_part)
    return out


# speedup vs baseline: 2.3784x; 1.0284x over previous
import jax
import jax.numpy as jnp
from jax.experimental import pallas as pl
from jax.experimental.pallas import tpu as pltpu

SCALE = 28.0
ALPHA = 0.6
BETA = 0.4
SCALE_POS = 10.0
SCALE_NEG = 40.0
NUM_CLASSES = 11003
FEATURE_SIZE = 2048
BATCH = 1024

_CB = 512
_NJ = 22
_E8 = 16.0
_CC = SCALE / (_E8 * _E8)


def _norm_body(e_ref, o8_ref):
    x = e_ref[...]
    n = x * jax.lax.rsqrt(jnp.sum(x * x, axis=1, keepdims=True))
    o8_ref[...] = (n * _E8).astype(jnp.float8_e4m3fn)


def _main_body(en_ref, w_ref, labb_ref, se_ref, la_ref, raw_ref):
    j = pl.program_id(0)

    @pl.when(j == 0)
    def _():
        se_ref[...] = jnp.zeros(se_ref.shape, se_ref.dtype)
        la_ref[...] = jnp.zeros(la_ref.shape, la_ref.dtype)

    w = w_ref[...]
    ssq = jnp.sum(w * w, axis=0, keepdims=True)
    rinv = _E8 * jax.lax.rsqrt(jnp.maximum(ssq, 1e-30))
    w8 = (w * rinv).astype(jnp.float8_e4m3fn)

    col0 = j * _CB
    cid1 = col0 + jax.lax.broadcasted_iota(jnp.int32, (1, _CB), 1)
    ok = cid1 < NUM_CLASSES

    raw_ref[...] = jnp.dot(en_ref[...], w8,
                           preferred_element_type=jnp.float32)

    raw = raw_ref[...]
    ex = jnp.exp(jnp.where(ok, raw * _CC - SCALE, -1e3))
    cid = col0 + jax.lax.broadcasted_iota(jnp.int32, (2 * BATCH, _CB), 1)
    labm = pltpu.repeat(labb_ref[...], _CB // 128, axis=1) == cid
    lv = jnp.where(labm, raw, 0.0)

    se_ref[0] += (ex[:, 0:128] + ex[:, 128:256]) + (ex[:, 256:384] + ex[:, 384:512])
    la_ref[0] += (lv[:, 0:128] + lv[:, 128:256]) + (lv[:, 256:384] + lv[:, 384:512])


def _sim_body(v_ref, t_ref, labv_ref, labr_ref, ga_ref):
    j = pl.program_id(0)

    @pl.when(j == 0)
    def _():
        ga_ref[...] = jnp.zeros(ga_ref.shape, ga_ref.dtype)

    raws = jax.lax.dot_general(v_ref[...], t_ref[...],
                               (((1,), (1,)), ((), ())),
                               preferred_element_type=jnp.float32)
    posm = pltpu.repeat(labv_ref[...], BATCH // 128, axis=1) == \
        jnp.broadcast_to(labr_ref[...], raws.shape)
    coef = jnp.where(posm, -SCALE_POS / (_E8 * _E8), SCALE_NEG / (_E8 * _E8))
    off = jnp.where(posm, SCALE_POS * ALPHA, -SCALE_NEG * BETA)
    x = coef * raws + off
    pp = jnp.maximum(x, 0.0) + jnp.log1p(jnp.exp(-jnp.abs(x)))
    prow = jnp.sum(pp, axis=0, keepdims=True)
    acc = ga_ref[0]
    for k in range(BATCH // 128):
        acc = acc + prow[:, k * 128:(k + 1) * 128]
    ga_ref[0] = acc


def _final_body(se_ref, la_ref, ga_ref, o_ref):
    s = se_ref[0]
    srow = jnp.sum(s, axis=1, keepdims=True)
    suml = jnp.sum(jnp.log(srow))
    labt = jnp.sum(la_ref[0])
    inst = (suml - _CC * labt) / BATCH + 2.0 * SCALE
    ga = 2.0 * jnp.sum(ga_ref[...]) / BATCH
    o_ref[0] = inst
    o_ref[1] = ga


def kernel(visual_embed, textual_embed, labels, W):
    labels = labels.astype(jnp.int32)
    E = jnp.concatenate([visual_embed, textual_embed], axis=0)
    lab2 = jnp.concatenate([labels, labels], axis=0)
    labb = jnp.broadcast_to(lab2[:, None], (2 * BATCH, 128))
    labv = jnp.broadcast_to(labels[:, None], (BATCH, 128))
    labr = labels[None, :]

    En8 = pl.pallas_call(
        _norm_body,
        grid=(2,),
        in_specs=[pl.BlockSpec((BATCH, FEATURE_SIZE), lambda i: (i, 0))],
        out_specs=pl.BlockSpec((BATCH, FEATURE_SIZE), lambda i: (i, 0)),
        out_shape=jax.ShapeDtypeStruct((2 * BATCH, FEATURE_SIZE),
                                       jnp.float8_e4m3fn),
        name="normalize_embed",
    )(E)

    se, la = pl.pallas_call(
        _main_body,
        grid=(_NJ,),
        in_specs=[
            pl.BlockSpec((2 * BATCH, FEATURE_SIZE), lambda j: (0, 0)),
            pl.BlockSpec((FEATURE_SIZE, _CB), lambda j: (0, j)),
            pl.BlockSpec((2 * BATCH, 128), lambda j: (0, 0)),
        ],
        out_specs=[
            pl.BlockSpec((1, 2 * BATCH, 128), lambda j: (0, 0, 0)),
            pl.BlockSpec((1, 2 * BATCH, 128), lambda j: (0, 0, 0)),
        ],
        out_shape=[
            jax.ShapeDtypeStruct((1, 2 * BATCH, 128), jnp.float32),
            jax.ShapeDtypeStruct((1, 2 * BATCH, 128), jnp.float32),
        ],
        scratch_shapes=[pltpu.VMEM((2 * BATCH, _CB), jnp.float32)],
        compiler_params=pltpu.CompilerParams(
            dimension_semantics=("arbitrary",),
            vmem_limit_bytes=100 * 1024 * 1024),
        name="cosine_ce_partials",
    )(En8, W, labb)

    ga_part = pl.pallas_call(
        _sim_body,
        grid=(8,),
        in_specs=[
            pl.BlockSpec((128, FEATURE_SIZE), lambda j: (j, 0)),
            pl.BlockSpec((BATCH, FEATURE_SIZE), lambda j: (1, 0)),
            pl.BlockSpec((128, 128), lambda j: (j, 0)),
            pl.BlockSpec((1, BATCH), lambda j: (0, 0)),
        ],
        out_specs=pl.BlockSpec((1, 1, 128), lambda j: (0, 0, 0)),
        out_shape=jax.ShapeDtypeStruct((1, 1, 128), jnp.float32),
        compiler_params=pltpu.CompilerParams(
            dimension_semantics=("arbitrary",),
            vmem_limit_bytes=100 * 1024 * 1024),
        name="global_align_partials",
    )(En8, En8, labv, labr)

    out = pl.pallas_call(
        _final_body,
        in_specs=[
            pl.BlockSpec(memory_space=pltpu.VMEM),
            pl.BlockSpec(memory_space=pltpu.VMEM),
            pl.BlockSpec(memory_space=pltpu.VMEM),
        ],
        out_specs=pl.BlockSpec(memory_space=pltpu.SMEM),
        out_shape=jax.ShapeDtypeStruct((2,), jnp.float32),
        name="loss_combine",
    )(se, la, ga_part)
    return out


# speedup vs baseline: 2.6981x; 1.1344x over previous
import jax
import jax.numpy as jnp
from jax.experimental import pallas as pl
from jax.experimental.pallas import tpu as pltpu

SCALE = 28.0
ALPHA = 0.6
BETA = 0.4
SCALE_POS = 10.0
SCALE_NEG = 40.0
NUM_CLASSES = 11003
FEATURE_SIZE = 2048
BATCH = 1024

_CB = 512
_NJ = 22
_E8 = 16.0
_CC = SCALE / (_E8 * _E8)


def _main_body(v_ref, t_ref, w_ref, labb_ref, se_ref, la_ref, en8_ref,
               raw_ref):
    j = pl.program_id(0)

    @pl.when(j == 0)
    def _():
        x = v_ref[...]
        n = x * jax.lax.rsqrt(jnp.sum(x * x, axis=1, keepdims=True))
        en8_ref[0:BATCH, :] = (n * _E8).astype(jnp.float8_e4m3fn)

    @pl.when(j == 1)
    def _():
        x = t_ref[...]
        n = x * jax.lax.rsqrt(jnp.sum(x * x, axis=1, keepdims=True))
        en8_ref[BATCH:2 * BATCH, :] = (n * _E8).astype(jnp.float8_e4m3fn)

    @pl.when(j == 2)
    def _():
        se_ref[...] = jnp.zeros(se_ref.shape, se_ref.dtype)
        la_ref[...] = jnp.zeros(la_ref.shape, la_ref.dtype)

    @pl.when(j >= 2)
    def _():
        w = w_ref[...]
        ssq = jnp.sum(w * w, axis=0, keepdims=True)
        rinv = _E8 * jax.lax.rsqrt(jnp.maximum(ssq, 1e-30))
        w8 = (w * rinv).astype(jnp.float8_e4m3fn)

        col0 = (j - 2) * _CB
        cid1 = col0 + jax.lax.broadcasted_iota(jnp.int32, (1, _CB), 1)
        ok = cid1 < NUM_CLASSES

        raw_ref[...] = jnp.dot(en8_ref[...], w8,
                               preferred_element_type=jnp.float32)

        raw = raw_ref[...]
        ex = jnp.exp(jnp.where(ok, raw * _CC - SCALE, -1e3))
        cid = col0 + jax.lax.broadcasted_iota(jnp.int32, (2 * BATCH, _CB), 1)
        labm = pltpu.repeat(labb_ref[...], _CB // 128, axis=1) == cid
        lv = jnp.where(labm, raw, 0.0)

        se_ref[0] += (ex[:, 0:128] + ex[:, 128:256]) + \
            (ex[:, 256:384] + ex[:, 384:512])
        la_ref[0] += (lv[:, 0:128] + lv[:, 128:256]) + \
            (lv[:, 256:384] + lv[:, 384:512])


def _sim_body(t8_ref, v8_ref, labv_ref, labr_ref, se_ref, la_ref,
              o_ref, ga_ref):
    j = pl.program_id(0)

    @pl.when(j == 0)
    def _():
        ga_ref[...] = jnp.zeros(ga_ref.shape, ga_ref.dtype)

    raws = jax.lax.dot_general(t8_ref[...], v8_ref[...],
                               (((1,), (1,)), ((), ())),
                               preferred_element_type=jnp.float32)
    posm = pltpu.repeat(labv_ref[...], BATCH // 128, axis=1) == \
        jnp.broadcast_to(labr_ref[...], raws.shape)
    coef = jnp.where(posm, -SCALE_POS / (_E8 * _E8), SCALE_NEG / (_E8 * _E8))
    off = jnp.where(posm, SCALE_POS * ALPHA, -SCALE_NEG * BETA)
    x = coef * raws + off
    pp = jnp.maximum(x, 0.0) + jnp.log1p(jnp.exp(-jnp.abs(x)))
    prow = jnp.sum(pp, axis=0, keepdims=True)
    acc = ga_ref[...]
    for k in range(BATCH // 128):
        acc = acc + prow[:, k * 128:(k + 1) * 128]
    ga_ref[...] = acc

    @pl.when(j == 1)
    def _():
        s = se_ref[0]
        srow = jnp.sum(s, axis=1, keepdims=True)
        suml = jnp.sum(jnp.log(srow))
        labt = jnp.sum(la_ref[0])
        inst = (suml - _CC * labt) / BATCH + 2.0 * SCALE
        ga = 2.0 * jnp.sum(ga_ref[...]) / BATCH
        o_ref[0] = inst
        o_ref[1] = ga


def kernel(visual_embed, textual_embed, labels, W):
    labels = labels.astype(jnp.int32)
    lab2 = jnp.concatenate([labels, labels], axis=0)
    labb = jnp.broadcast_to(lab2[:, None], (2 * BATCH, 128))
    labv = jnp.broadcast_to(labels[:, None], (BATCH, 128))
    labr = labels[None, :]

    se, la, En8 = pl.pallas_call(
        _main_body,
        grid=(_NJ + 2,),
        in_specs=[
            pl.BlockSpec((BATCH, FEATURE_SIZE), lambda j: (0, 0)),
            pl.BlockSpec((BATCH, FEATURE_SIZE), lambda j: (0, 0)),
            pl.BlockSpec((FEATURE_SIZE, _CB),
                         lambda j: (0, jnp.maximum(j - 2, 0))),
            pl.BlockSpec((2 * BATCH, 128), lambda j: (0, 0)),
        ],
        out_specs=[
            pl.BlockSpec((1, 2 * BATCH, 128), lambda j: (0, 0, 0)),
            pl.BlockSpec((1, 2 * BATCH, 128), lambda j: (0, 0, 0)),
            pl.BlockSpec((2 * BATCH, FEATURE_SIZE), lambda j: (0, 0)),
        ],
        out_shape=[
            jax.ShapeDtypeStruct((1, 2 * BATCH, 128), jnp.float32),
            jax.ShapeDtypeStruct((1, 2 * BATCH, 128), jnp.float32),
            jax.ShapeDtypeStruct((2 * BATCH, FEATURE_SIZE),
                                 jnp.float8_e4m3fn),
        ],
        scratch_shapes=[pltpu.VMEM((2 * BATCH, _CB), jnp.float32)],
        compiler_params=pltpu.CompilerParams(
            dimension_semantics=("arbitrary",),
            vmem_limit_bytes=100 * 1024 * 1024),
        name="cosine_ce",
    )(visual_embed, textual_embed, W, labb)

    out = pl.pallas_call(
        _sim_body,
        grid=(2,),
        in_specs=[
            pl.BlockSpec((BATCH // 2, FEATURE_SIZE), lambda j: (j + 2, 0)),
            pl.BlockSpec((BATCH, FEATURE_SIZE), lambda j: (0, 0)),
            pl.BlockSpec((BATCH // 2, 128), lambda j: (j, 0)),
            pl.BlockSpec((1, BATCH), lambda j: (0, 0)),
            pl.BlockSpec((1, 2 * BATCH, 128), lambda j: (0, 0, 0)),
            pl.BlockSpec((1, 2 * BATCH, 128), lambda j: (0, 0, 0)),
        ],
        out_specs=pl.BlockSpec(memory_space=pltpu.SMEM),
        out_shape=jax.ShapeDtypeStruct((2,), jnp.float32),
        scratch_shapes=[pltpu.VMEM((1, 128), jnp.float32)],
        compiler_params=pltpu.CompilerParams(
            dimension_semantics=("arbitrary",),
            vmem_limit_bytes=100 * 1024 * 1024),
        name="global_align",
    )(En8, En8, labv, labr, se, la)
    return out


# speedup vs baseline: 3.1435x; 1.1651x over previous
import jax
import jax.numpy as jnp
from jax.experimental import pallas as pl
from jax.experimental.pallas import tpu as pltpu

SCALE = 28.0
ALPHA = 0.6
BETA = 0.4
SCALE_POS = 10.0
SCALE_NEG = 40.0
NUM_CLASSES = 11003
FEATURE_SIZE = 2048
BATCH = 1024

_CB = 512
_MC = 256
_NJ = 22
_E8 = 16.0
_CC = SCALE / (_E8 * _E8)


def _main_body(v_ref, t_ref, w_ref, labb_ref, se_ref, la_ref, en8_ref):
    j = pl.program_id(0)

    @pl.when(j == 0)
    def _():
        x = v_ref[...]
        n = x * jax.lax.rsqrt(jnp.sum(x * x, axis=1, keepdims=True))
        en8_ref[0:BATCH, :] = (n * _E8).astype(jnp.float8_e4m3fn)

    @pl.when(j == 1)
    def _():
        x = t_ref[...]
        n = x * jax.lax.rsqrt(jnp.sum(x * x, axis=1, keepdims=True))
        en8_ref[BATCH:2 * BATCH, :] = (n * _E8).astype(jnp.float8_e4m3fn)

    @pl.when(j == 2)
    def _():
        se_ref[...] = jnp.zeros(se_ref.shape, se_ref.dtype)
        la_ref[...] = jnp.zeros(la_ref.shape, la_ref.dtype)

    @pl.when(j >= 2)
    def _():
        w = w_ref[...]
        ssq = jnp.sum(w * w, axis=0, keepdims=True)
        rinv = _E8 * jax.lax.rsqrt(jnp.maximum(ssq, 1e-30))
        w8 = (w * rinv).astype(jnp.float8_e4m3fn)

        col0 = (j - 2) * _CB
        cid1 = col0 + jax.lax.broadcasted_iota(jnp.int32, (1, _CB), 1)
        ok = cid1 < NUM_CLASSES

        cid = col0 + jax.lax.broadcasted_iota(jnp.int32, (_MC, _CB), 1)
        for r in range(0, 2 * BATCH, _MC):
            raw = jnp.dot(en8_ref[r:r + _MC, :], w8,
                          preferred_element_type=jnp.float32)
            ex = jnp.exp(jnp.where(ok, raw * _CC - SCALE, -1e3))
            labm = pltpu.repeat(labb_ref[r:r + _MC, :],
                                _CB // 128, axis=1) == cid
            lv = jnp.where(labm, raw, 0.0)
            se_ref[0, r:r + _MC, :] += (ex[:, 0:128] + ex[:, 128:256]) + \
                (ex[:, 256:384] + ex[:, 384:512])
            la_ref[0, r:r + _MC, :] += (lv[:, 0:128] + lv[:, 128:256]) + \
                (lv[:, 256:384] + lv[:, 384:512])


def _sim_body(t8_ref, v8_ref, labv_ref, labr_ref, se_ref, la_ref,
              o_ref, ga_ref):
    j = pl.program_id(0)

    @pl.when(j == 0)
    def _():
        ga_ref[...] = jnp.zeros(ga_ref.shape, ga_ref.dtype)

    raws = jax.lax.dot_general(t8_ref[...], v8_ref[...],
                               (((1,), (1,)), ((), ())),
                               preferred_element_type=jnp.float32)
    posm = pltpu.repeat(labv_ref[...], BATCH // 128, axis=1) == \
        jnp.broadcast_to(labr_ref[...], raws.shape)
    coef = jnp.where(posm, -SCALE_POS / (_E8 * _E8), SCALE_NEG / (_E8 * _E8))
    off = jnp.where(posm, SCALE_POS * ALPHA, -SCALE_NEG * BETA)
    x = coef * raws + off
    pp = jnp.maximum(x, 0.0) + jnp.log1p(jnp.exp(-jnp.abs(x)))
    prow = jnp.sum(pp, axis=0, keepdims=True)
    acc = ga_ref[...]
    for k in range(BATCH // 128):
        acc = acc + prow[:, k * 128:(k + 1) * 128]
    ga_ref[...] = acc

    @pl.when(j == 1)
    def _():
        s = se_ref[0]
        srow = jnp.sum(s, axis=1, keepdims=True)
        suml = jnp.sum(jnp.log(srow))
        labt = jnp.sum(la_ref[0])
        inst = (suml - _CC * labt) / BATCH + 2.0 * SCALE
        ga = 2.0 * jnp.sum(ga_ref[...]) / BATCH
        o_ref[0] = inst
        o_ref[1] = ga


def kernel(visual_embed, textual_embed, labels, W):
    labels = labels.astype(jnp.int32)
    lab2 = jnp.concatenate([labels, labels], axis=0)
    labb = jnp.broadcast_to(lab2[:, None], (2 * BATCH, 128))
    labv = jnp.broadcast_to(labels[:, None], (BATCH, 128))
    labr = labels[None, :]

    se, la, En8 = pl.pallas_call(
        _main_body,
        grid=(_NJ + 2,),
        in_specs=[
            pl.BlockSpec((BATCH, FEATURE_SIZE), lambda j: (0, 0)),
            pl.BlockSpec((BATCH, FEATURE_SIZE), lambda j: (0, 0)),
            pl.BlockSpec((FEATURE_SIZE, _CB),
                         lambda j: (0, jnp.maximum(j - 2, 0))),
            pl.BlockSpec((2 * BATCH, 128), lambda j: (0, 0)),
        ],
        out_specs=[
            pl.BlockSpec((1, 2 * BATCH, 128), lambda j: (0, 0, 0)),
            pl.BlockSpec((1, 2 * BATCH, 128), lambda j: (0, 0, 0)),
            pl.BlockSpec((2 * BATCH, FEATURE_SIZE), lambda j: (0, 0)),
        ],
        out_shape=[
            jax.ShapeDtypeStruct((1, 2 * BATCH, 128), jnp.float32),
            jax.ShapeDtypeStruct((1, 2 * BATCH, 128), jnp.float32),
            jax.ShapeDtypeStruct((2 * BATCH, FEATURE_SIZE),
                                 jnp.float8_e4m3fn),
        ],
        compiler_params=pltpu.CompilerParams(
            dimension_semantics=("arbitrary",),
            vmem_limit_bytes=100 * 1024 * 1024),
        name="cosine_ce",
    )(visual_embed, textual_embed, W, labb)

    out = pl.pallas_call(
        _sim_body,
        grid=(2,),
        in_specs=[
            pl.BlockSpec((BATCH // 2, FEATURE_SIZE), lambda j: (j + 2, 0)),
            pl.BlockSpec((BATCH, FEATURE_SIZE), lambda j: (0, 0)),
            pl.BlockSpec((BATCH // 2, 128), lambda j: (j, 0)),
            pl.BlockSpec((1, BATCH), lambda j: (0, 0)),
            pl.BlockSpec((1, 2 * BATCH, 128), lambda j: (0, 0, 0)),
            pl.BlockSpec((1, 2 * BATCH, 128), lambda j: (0, 0, 0)),
        ],
        out_specs=pl.BlockSpec(memory_space=pltpu.SMEM),
        out_shape=jax.ShapeDtypeStruct((2,), jnp.float32),
        scratch_shapes=[pltpu.VMEM((1, 128), jnp.float32)],
        compiler_params=pltpu.CompilerParams(
            dimension_semantics=("arbitrary",),
            vmem_limit_bytes=100 * 1024 * 1024),
        name="global_align",
    )(En8, En8, labv, labr, se, la)
    return out


# speedup vs baseline: 3.3056x; 1.0515x over previous
import jax
import jax.numpy as jnp
from jax.experimental import pallas as pl
from jax.experimental.pallas import tpu as pltpu

SCALE = 28.0
ALPHA = 0.6
BETA = 0.4
SCALE_POS = 10.0
SCALE_NEG = 40.0
NUM_CLASSES = 11003
FEATURE_SIZE = 2048
BATCH = 1024

_CB = 1024
_MC = 256
_NJ = 11
_E8 = 16.0
_CC = SCALE / (_E8 * _E8)


def _main_body(v_ref, t_ref, w_ref, labb_ref, se_ref, la_ref, en8_ref):
    j = pl.program_id(0)

    @pl.when(j == 0)
    def _():
        x = v_ref[...]
        n = x * jax.lax.rsqrt(jnp.sum(x * x, axis=1, keepdims=True))
        en8_ref[0:BATCH, :] = (n * _E8).astype(jnp.float8_e4m3fn)

    @pl.when(j == 1)
    def _():
        x = t_ref[...]
        n = x * jax.lax.rsqrt(jnp.sum(x * x, axis=1, keepdims=True))
        en8_ref[BATCH:2 * BATCH, :] = (n * _E8).astype(jnp.float8_e4m3fn)

    @pl.when(j == 2)
    def _():
        se_ref[...] = jnp.zeros(se_ref.shape, se_ref.dtype)
        la_ref[...] = jnp.zeros(la_ref.shape, la_ref.dtype)

    @pl.when(j >= 2)
    def _():
        w = w_ref[...]
        ssq = jnp.sum(w * w, axis=0, keepdims=True)
        rinv = _E8 * jax.lax.rsqrt(jnp.maximum(ssq, 1e-30))
        w8 = (w * rinv).astype(jnp.float8_e4m3fn)

        col0 = (j - 2) * _CB
        cid1 = col0 + jax.lax.broadcasted_iota(jnp.int32, (1, _CB), 1)
        ok = cid1 < NUM_CLASSES

        cid = col0 + jax.lax.broadcasted_iota(jnp.int32, (_MC, _CB), 1)
        for r in range(0, 2 * BATCH, _MC):
            raw = jnp.dot(en8_ref[r:r + _MC, :], w8,
                          preferred_element_type=jnp.float32)
            ex = jnp.exp(jnp.where(ok, raw * _CC - SCALE, -1e3))
            labm = pltpu.repeat(labb_ref[r:r + _MC, :],
                                _CB // 128, axis=1) == cid
            lv = jnp.where(labm, raw, 0.0)
            exs = [ex[:, k * 128:(k + 1) * 128] for k in range(_CB // 128)]
            lvs = [lv[:, k * 128:(k + 1) * 128] for k in range(_CB // 128)]
            while len(exs) > 1:
                exs = [a + b for a, b in zip(exs[::2], exs[1::2])]
                lvs = [a + b for a, b in zip(lvs[::2], lvs[1::2])]
            se_ref[0, r:r + _MC, :] += exs[0]
            la_ref[0, r:r + _MC, :] += lvs[0]


def _sim_body(t8_ref, v8_ref, labv_ref, labr_ref, se_ref, la_ref,
              o_ref, ga_ref):
    j = pl.program_id(0)

    @pl.when(j == 0)
    def _():
        ga_ref[...] = jnp.zeros(ga_ref.shape, ga_ref.dtype)

    raws = jax.lax.dot_general(t8_ref[...], v8_ref[...],
                               (((1,), (1,)), ((), ())),
                               preferred_element_type=jnp.float32)
    posm = pltpu.repeat(labv_ref[...], BATCH // 128, axis=1) == \
        jnp.broadcast_to(labr_ref[...], raws.shape)
    coef = jnp.where(posm, -SCALE_POS / (_E8 * _E8), SCALE_NEG / (_E8 * _E8))
    off = jnp.where(posm, SCALE_POS * ALPHA, -SCALE_NEG * BETA)
    x = coef * raws + off
    pp = jnp.maximum(x, 0.0) + jnp.log1p(jnp.exp(-jnp.abs(x)))
    prow = jnp.sum(pp, axis=0, keepdims=True)
    acc = ga_ref[...]
    for k in range(BATCH // 128):
        acc = acc + prow[:, k * 128:(k + 1) * 128]
    ga_ref[...] = acc

    @pl.when(j == 1)
    def _():
        s = se_ref[0]
        srow = jnp.sum(s, axis=1, keepdims=True)
        suml = jnp.sum(jnp.log(srow))
        labt = jnp.sum(la_ref[0])
        inst = (suml - _CC * labt) / BATCH + 2.0 * SCALE
        ga = 2.0 * jnp.sum(ga_ref[...]) / BATCH
        o_ref[0] = inst
        o_ref[1] = ga


def kernel(visual_embed, textual_embed, labels, W):
    labels = labels.astype(jnp.int32)
    lab2 = jnp.concatenate([labels, labels], axis=0)
    labb = jnp.broadcast_to(lab2[:, None], (2 * BATCH, 128))
    labv = jnp.broadcast_to(labels[:, None], (BATCH, 128))
    labr = labels[None, :]

    se, la, En8 = pl.pallas_call(
        _main_body,
        grid=(_NJ + 2,),
        in_specs=[
            pl.BlockSpec((BATCH, FEATURE_SIZE), lambda j: (0, 0)),
            pl.BlockSpec((BATCH, FEATURE_SIZE), lambda j: (0, 0)),
            pl.BlockSpec((FEATURE_SIZE, _CB),
                         lambda j: (0, jnp.maximum(j - 2, 0))),
            pl.BlockSpec((2 * BATCH, 128), lambda j: (0, 0)),
        ],
        out_specs=[
            pl.BlockSpec((1, 2 * BATCH, 128), lambda j: (0, 0, 0)),
            pl.BlockSpec((1, 2 * BATCH, 128), lambda j: (0, 0, 0)),
            pl.BlockSpec((2 * BATCH, FEATURE_SIZE), lambda j: (0, 0)),
        ],
        out_shape=[
            jax.ShapeDtypeStruct((1, 2 * BATCH, 128), jnp.float32),
            jax.ShapeDtypeStruct((1, 2 * BATCH, 128), jnp.float32),
            jax.ShapeDtypeStruct((2 * BATCH, FEATURE_SIZE),
                                 jnp.float8_e4m3fn),
        ],
        compiler_params=pltpu.CompilerParams(
            dimension_semantics=("arbitrary",),
            vmem_limit_bytes=100 * 1024 * 1024),
        name="cosine_ce",
    )(visual_embed, textual_embed, W, labb)

    out = pl.pallas_call(
        _sim_body,
        grid=(2,),
        in_specs=[
            pl.BlockSpec((BATCH // 2, FEATURE_SIZE), lambda j: (j + 2, 0)),
            pl.BlockSpec((BATCH, FEATURE_SIZE), lambda j: (0, 0)),
            pl.BlockSpec((BATCH // 2, 128), lambda j: (j, 0)),
            pl.BlockSpec((1, BATCH), lambda j: (0, 0)),
            pl.BlockSpec((1, 2 * BATCH, 128), lambda j: (0, 0, 0)),
            pl.BlockSpec((1, 2 * BATCH, 128), lambda j: (0, 0, 0)),
        ],
        out_specs=pl.BlockSpec(memory_space=pltpu.SMEM),
        out_shape=jax.ShapeDtypeStruct((2,), jnp.float32),
        scratch_shapes=[pltpu.VMEM((1, 128), jnp.float32)],
        compiler_params=pltpu.CompilerParams(
            dimension_semantics=("arbitrary",),
            vmem_limit_bytes=100 * 1024 * 1024),
        name="global_align",
    )(En8, En8, labv, labr, se, la)
    return out
